# baseline (speedup 1.0000x reference)
import jax
import jax.numpy as jnp
from jax.experimental import pallas as pl
from jax.experimental.pallas import tpu as pltpu

_BT = 512
_NBUF = 6


def _gate_kernel(x_hbm, w_ref, o_ref, xbuf, sems):
    i = pl.program_id(0)
    nsteps = pl.num_programs(0)

    def copy_in(slot, blk):
        return pltpu.make_async_copy(
            x_hbm.at[pl.ds(blk * _BT, _BT), :], xbuf.at[slot], sems.at[slot])

    @pl.when(i == 0)
    def _():
        for b in range(_NBUF):
            copy_in(b, b).start()

    slot = jax.lax.rem(i, _NBUF)
    copy_in(slot, i).wait()
    o_ref[...] = jax.lax.dot_general(
        xbuf[slot], w_ref[...],
        dimension_numbers=(((1,), (1,)), ((), ())),
        preferred_element_type=jnp.float32,
    )

    nxt = i + _NBUF

    @pl.when(nxt < nsteps)
    def _():
        copy_in(slot, nxt).start()


def kernel(x, wg_weight):
    tokens, model_dim = x.shape
    num_experts = wg_weight.shape[0]
    return pl.pallas_call(
        _gate_kernel,
        grid=(tokens // _BT,),
        in_specs=[
            pl.BlockSpec(memory_space=pl.ANY),
            pl.BlockSpec((num_experts, model_dim), lambda i: (0, 0)),
        ],
        out_specs=pl.BlockSpec((_BT, num_experts), lambda i: (i, 0)),
        out_shape=jax.ShapeDtypeStruct((tokens, num_experts), jnp.float32),
        scratch_shapes=[
            pltpu.VMEM((_NBUF, _BT, model_dim), jnp.float32),
            pltpu.SemaphoreType.DMA((_NBUF,)),
        ],
    )(x, wg_weight)


# speedup vs baseline: 1.0380x; 1.0380x over previous
import jax
import jax.numpy as jnp
from jax.experimental import pallas as pl

_BT = 512


def _gate_kernel(xa_ref, xb_ref, w_ref, o_ref):
    w = w_ref[...]
    dims = (((1,), (1,)), ((), ()))
    o_ref[:_BT, :] = jax.lax.dot_general(
        xa_ref[...], w, dimension_numbers=dims,
        preferred_element_type=jnp.float32)
    o_ref[_BT:, :] = jax.lax.dot_general(
        xb_ref[...], w, dimension_numbers=dims,
        preferred_element_type=jnp.float32)


def kernel(x, wg_weight):
    tokens, model_dim = x.shape
    num_experts = wg_weight.shape[0]
    return pl.pallas_call(
        _gate_kernel,
        grid=(tokens // (2 * _BT),),
        in_specs=[
            pl.BlockSpec((_BT, model_dim), lambda i: (2 * i, 0)),
            pl.BlockSpec((_BT, model_dim), lambda i: (2 * i + 1, 0)),
            pl.BlockSpec((num_experts, model_dim), lambda i: (0, 0)),
        ],
        out_specs=pl.BlockSpec((2 * _BT, num_experts), lambda i: (i, 0)),
        out_shape=jax.ShapeDtypeStruct((tokens, num_experts), jnp.float32),
    )(x, x, wg_weight)


# speedup vs baseline: 1.0512x; 1.0126x over previous
import jax
import jax.numpy as jnp
from jax.experimental import pallas as pl
from jax.experimental.pallas import tpu as pltpu

_BT = 512


def _gate_kernel(x_hbm, w_ref, o_ref, xa, xb, sa, sb):
    i = pl.program_id(0)
    n = pl.num_programs(0)

    def cp_a(slot, blk):
        return pltpu.make_async_copy(
            x_hbm.at[pl.ds(blk * _BT, _BT), :], xa.at[slot], sa.at[slot])

    def cp_b(slot, blk):
        return pltpu.make_async_copy(
            x_hbm.at[pl.ds(blk * _BT, _BT), :], xb.at[slot], sb.at[slot])

    @pl.when(i == 0)
    def _():
        cp_a(0, 0).start()
        cp_b(0, 1).start()
        cp_a(1, 2).start()
        cp_b(1, 3).start()

    slot = jax.lax.rem(i, 2)
    w = w_ref[...]
    dims = (((1,), (1,)), ((), ()))
    nxt = i + 2

    cp_a(slot, 2 * i).wait()
    o_ref[:_BT, :] = jax.lax.dot_general(
        xa[slot], w, dimension_numbers=dims,
        preferred_element_type=jnp.float32)

    @pl.when(nxt < n)
    def _():
        cp_a(slot, 2 * nxt).start()

    cp_b(slot, 2 * i + 1).wait()
    o_ref[_BT:, :] = jax.lax.dot_general(
        xb[slot], w, dimension_numbers=dims,
        preferred_element_type=jnp.float32)

    @pl.when(nxt < n)
    def _():
        cp_b(slot, 2 * nxt + 1).start()


def kernel(x, wg_weight):
    tokens, model_dim = x.shape
    num_experts = wg_weight.shape[0]
    return pl.pallas_call(
        _gate_kernel,
        grid=(tokens // (2 * _BT),),
        in_specs=[
            pl.BlockSpec(memory_space=pl.ANY),
            pl.BlockSpec((num_experts, model_dim), lambda i: (0, 0)),
        ],
        out_specs=pl.BlockSpec((2 * _BT, num_experts), lambda i: (i, 0)),
        out_shape=jax.ShapeDtypeStruct((tokens, num_experts), jnp.float32),
        scratch_shapes=[
            pltpu.VMEM((2, _BT, model_dim), jnp.float32),
            pltpu.VMEM((2, _BT, model_dim), jnp.float32),
            pltpu.SemaphoreType.DMA((2,)),
            pltpu.SemaphoreType.DMA((2,)),
        ],
    )(x, wg_weight)
